# baseline (speedup 1.0000x reference)
import numpy as np
import jax
import jax.numpy as jnp
from jax.experimental import pallas as pl
from jax.experimental.pallas import tpu as pltpu


def _box_w_matrix_t(n: int) -> np.ndarray:
    idx = np.arange(n)
    taps = (np.abs(idx[:, None] - idx[None, :]) <= 1).astype(np.float32)
    return np.ascontiguousarray((taps / taps.sum(axis=1, keepdims=True)).T)


def _resize_matrix(out_n: int, in_n: int) -> np.ndarray:
    if out_n == 1:
        m = np.zeros((1, in_n), np.float32)
        m[0, 0] = 1.0
        return m
    src = np.arange(out_n, dtype=np.float32) * (in_n - 1) / (out_n - 1)
    lo = np.clip(np.floor(src).astype(np.int32), 0, in_n - 1)
    hi = np.minimum(lo + 1, in_n - 1)
    t = src - lo.astype(np.float32)
    m = np.zeros((out_n, in_n), np.float32)
    np.add.at(m, (np.arange(out_n), lo), 1.0 - t)
    np.add.at(m, (np.arange(out_n), hi), t)
    return m


def _coeff_planes(x, y, swn_t, uw_t, w1, s1, b1, w2, s2, b2, w3, wh):
    hl, wl = x.shape[1], x.shape[2]
    p12 = jnp.concatenate([x, y, x * y, x * x], axis=0).reshape(12 * hl, wl)

    qw = jnp.dot(p12, swn_t, preferred_element_type=jnp.float32)
    r = jax.lax.broadcasted_iota(jnp.int32, qw.shape, 0) & (hl - 1)
    dn = pltpu.roll(qw, 1, axis=0)
    up = pltpu.roll(qw, 12 * hl - 1, axis=0)
    ssum = (qw + jnp.where(r == 0, 0.0, dn)
            + jnp.where(r == hl - 1, 0.0, up))
    nh = jnp.where(r == 0, 0.5, jnp.where(r == hl - 1, 0.5, 1.0 / 3.0))
    box3 = (ssum * nh).reshape(12, hl, wl)

    mx = box3[0:3]
    my = box3[3:6]
    cov = box3[6:9] - mx * my
    var = box3[9:12] - mx * mx

    feats = jnp.concatenate([cov, var], axis=0).reshape(6, hl * wl)
    h = jnp.dot(w1, feats, preferred_element_type=jnp.float32)
    h = jnp.maximum(h * s1 + b1, 0.0)
    h = jnp.dot(w2, h, preferred_element_type=jnp.float32)
    h = jnp.maximum(h * s2 + b2, 0.0)
    a = jnp.dot(w3, h, preferred_element_type=jnp.float32).reshape(3, hl, wl)
    bb = my - a * mx

    ab = jnp.concatenate([a, bb], axis=0).reshape(6 * hl, wl)
    return jnp.dot(ab, uw_t, preferred_element_type=jnp.float32)


def _gf_kernel(x_lr_ref, y_lr_ref, swn_t_ref, uw_t_ref, uh_ref,
               w1_ref, s1_ref, b1_ref, w2_ref, s2_ref, b2_ref, w3_ref,
               x_hr_hbm, out_hbm, planes_ref, xbuf, obuf, in_sem, out_sem):
    bpc = x_lr_ref.shape[0]
    hl = x_lr_ref.shape[2]
    hh, wh = out_hbm.shape[2], out_hbm.shape[3]
    core = pl.program_id(0)
    gb0 = core * bpc

    def in_copy(k):
        return pltpu.make_async_copy(
            x_hr_hbm.at[gb0 + k], xbuf.at[k % 2], in_sem.at[k % 2])

    def out_copy(k):
        return pltpu.make_async_copy(
            obuf.at[k % 2], out_hbm.at[gb0 + k], out_sem.at[k % 2])

    in_copy(0).start()
    if bpc > 1:
        in_copy(1).start()
    for k in range(bpc):
        m6 = _coeff_planes(x_lr_ref[k], y_lr_ref[k],
                           swn_t_ref[...], uw_t_ref[...],
                           w1_ref[...], s1_ref[...], b1_ref[...],
                           w2_ref[...], s2_ref[...], b2_ref[...],
                           w3_ref[...], wh)
        for c in range(3):
            planes_ref[k, :, (2 * c) * wh:(2 * c + 1) * wh] = \
                m6[c * hl:(c + 1) * hl]
            planes_ref[k, :, (2 * c + 1) * wh:(2 * c + 2) * wh] = \
                m6[(3 + c) * hl:(4 + c) * hl]

    uh_blk = uh_ref[...]
    for k in range(bpc):
        s = k % 2
        in_copy(k).wait()
        if k >= 2:
            out_copy(k - 2).wait()
        for c in range(3):
            pc = planes_ref[k, :, (2 * c) * wh:(2 * c + 2) * wh]
            m = jnp.dot(uh_blk, pc, preferred_element_type=jnp.float32)
            obuf[s, c] = m[:, :wh] * xbuf[s, c] + m[:, wh:]
        out_copy(k).start()
        if k + 2 < bpc:
            in_copy(k + 2).start()
    if bpc > 1:
        out_copy(bpc - 2).wait()
    out_copy(bpc - 1).wait()


def kernel(x_lr, y_lr, x_hr, w1, w2, w3, s1, b1, s2, b2):
    b, c, hl, wl = x_lr.shape
    _, _, hh, wh = x_hr.shape
    assert c == 3 and (hl & (hl - 1)) == 0 and b % 2 == 0

    swn_t = jnp.asarray(_box_w_matrix_t(wl))
    uw_t = jnp.asarray(_resize_matrix(wh, wl).T)
    uh = jnp.asarray(_resize_matrix(hh, hl))
    s1c, b1c = s1.reshape(-1, 1), b1.reshape(-1, 1)
    s2c, b2c = s2.reshape(-1, 1), b2.reshape(-1, 1)

    n_cores = 2
    bpc = b // n_cores

    cspec = lambda a: pl.BlockSpec(a.shape, lambda i: (0,) * a.ndim)
    hbm = pl.BlockSpec(memory_space=pl.ANY)

    flops = b * (2 * 12 * hl * wl * (hl + wl)
                 + 2 * hl * wl * (6 * 32 + 32 * 32 + 32 * 3)
                 + 2 * 6 * hl * wl * wh
                 + 2 * 6 * hh * hl * wh + 2 * 3 * hh * wh)
    bytes_accessed = b * 4 * (2 * 3 * hl * wl + 2 * 3 * hh * wh) + 4 * hh * hl

    return pl.pallas_call(
        _gf_kernel,
        out_shape=jax.ShapeDtypeStruct((b, 3, hh, wh), jnp.float32),
        grid=(n_cores,),
        in_specs=[
            pl.BlockSpec((bpc, 3, hl, wl), lambda i: (i, 0, 0, 0)),
            pl.BlockSpec((bpc, 3, hl, wl), lambda i: (i, 0, 0, 0)),
            cspec(swn_t), cspec(uw_t), cspec(uh),
            cspec(w1), cspec(s1c), cspec(b1c),
            cspec(w2), cspec(s2c), cspec(b2c),
            cspec(w3),
            hbm,
        ],
        out_specs=hbm,
        scratch_shapes=[
            pltpu.VMEM((bpc, hl, 6 * wh), jnp.float32),
            pltpu.VMEM((2, 3, hh, wh), jnp.float32),
            pltpu.VMEM((2, 3, hh, wh), jnp.float32),
            pltpu.SemaphoreType.DMA((2,)),
            pltpu.SemaphoreType.DMA((2,)),
        ],
        compiler_params=pltpu.CompilerParams(
            dimension_semantics=("parallel",),
            vmem_limit_bytes=57 * 1024 * 1024,
        ),
        cost_estimate=pl.CostEstimate(flops=flops, transcendentals=0,
                                      bytes_accessed=bytes_accessed),
    )(x_lr, y_lr, swn_t, uw_t, uh, w1, s1c, b1c, w2, s2c, b2c, w3, x_hr)


# speedup vs baseline: 1.2016x; 1.2016x over previous
import numpy as np
import jax
import jax.numpy as jnp
from jax.experimental import pallas as pl
from jax.experimental.pallas import tpu as pltpu


def _box_w_matrix_t(n: int) -> np.ndarray:
    idx = np.arange(n)
    taps = (np.abs(idx[:, None] - idx[None, :]) <= 1).astype(np.float32)
    return np.ascontiguousarray((taps / taps.sum(axis=1, keepdims=True)).T)


def _resize_matrix(out_n: int, in_n: int) -> np.ndarray:
    if out_n == 1:
        m = np.zeros((1, in_n), np.float32)
        m[0, 0] = 1.0
        return m
    src = np.arange(out_n, dtype=np.float32) * (in_n - 1) / (out_n - 1)
    lo = np.clip(np.floor(src).astype(np.int32), 0, in_n - 1)
    hi = np.minimum(lo + 1, in_n - 1)
    t = src - lo.astype(np.float32)
    m = np.zeros((out_n, in_n), np.float32)
    np.add.at(m, (np.arange(out_n), lo), 1.0 - t)
    np.add.at(m, (np.arange(out_n), hi), t)
    return m


def _coeff_planes(x, y, swn_t, uw_t, w1, s1, b1, w2, s2, b2, w3, wh):
    hl, wl = x.shape[1], x.shape[2]
    p12 = jnp.concatenate([x, y, x * y, x * x], axis=0).reshape(12 * hl, wl)

    qw = jnp.dot(p12, swn_t, preferred_element_type=jnp.float32)
    r = jax.lax.broadcasted_iota(jnp.int32, qw.shape, 0) & (hl - 1)
    dn = pltpu.roll(qw, 1, axis=0)
    up = pltpu.roll(qw, 12 * hl - 1, axis=0)
    ssum = (qw + jnp.where(r == 0, 0.0, dn)
            + jnp.where(r == hl - 1, 0.0, up))
    nh = jnp.where(r == 0, 0.5, jnp.where(r == hl - 1, 0.5, 1.0 / 3.0))
    box3 = (ssum * nh).reshape(12, hl, wl)

    mx = box3[0:3]
    my = box3[3:6]
    cov = box3[6:9] - mx * my
    var = box3[9:12] - mx * mx

    feats = jnp.concatenate([cov, var], axis=0).reshape(6, hl * wl)
    h = jnp.dot(w1, feats, preferred_element_type=jnp.float32)
    h = jnp.maximum(h * s1 + b1, 0.0)
    h = jnp.dot(w2, h, preferred_element_type=jnp.float32)
    h = jnp.maximum(h * s2 + b2, 0.0)
    a = jnp.dot(w3, h, preferred_element_type=jnp.float32).reshape(3, hl, wl)
    bb = my - a * mx

    ab = jnp.concatenate([a, bb], axis=0).reshape(6 * hl, wl)
    return jnp.dot(ab, uw_t, preferred_element_type=jnp.float32)


def _gf_kernel(x_lr_ref, y_lr_ref, swn_t_ref, uw_t_ref, uh_ref,
               w1_ref, s1_ref, b1_ref, w2_ref, s2_ref, b2_ref, w3_ref,
               x_hr_hbm, out_hbm, planes_ref, xbuf, obuf, in_sem, out_sem):
    bpc = x_lr_ref.shape[0]
    hl = x_lr_ref.shape[2]
    hh, wh = out_hbm.shape[2], out_hbm.shape[3]
    core = pl.program_id(0)
    gb0 = core * bpc

    in_depth = min(bpc, 4)

    def in_copy(k):
        return pltpu.make_async_copy(
            x_hr_hbm.at[gb0 + k], xbuf.at[k % in_depth], in_sem.at[k % in_depth])

    def out_copy(k):
        return pltpu.make_async_copy(
            obuf.at[k % 2], out_hbm.at[gb0 + k], out_sem.at[k % 2])

    for k in range(in_depth):
        in_copy(k).start()
    for k in range(bpc):
        m6 = _coeff_planes(x_lr_ref[k], y_lr_ref[k],
                           swn_t_ref[...], uw_t_ref[...],
                           w1_ref[...], s1_ref[...], b1_ref[...],
                           w2_ref[...], s2_ref[...], b2_ref[...],
                           w3_ref[...], wh)
        for c in range(3):
            planes_ref[k, :, (2 * c) * wh:(2 * c + 1) * wh] = \
                m6[c * hl:(c + 1) * hl]
            planes_ref[k, :, (2 * c + 1) * wh:(2 * c + 2) * wh] = \
                m6[(3 + c) * hl:(4 + c) * hl]

    uh_blk = uh_ref[...]
    for k in range(bpc):
        s = k % in_depth
        os = k % 2
        in_copy(k).wait()
        if k >= 2:
            out_copy(k - 2).wait()
        for c in range(3):
            pc = planes_ref[k, :, (2 * c) * wh:(2 * c + 2) * wh]
            m = jnp.dot(uh_blk, pc, preferred_element_type=jnp.float32)
            obuf[os, c] = m[:, :wh] * xbuf[s, c] + m[:, wh:]
        out_copy(k).start()
        if k + in_depth < bpc:
            in_copy(k + in_depth).start()
    if bpc > 1:
        out_copy(bpc - 2).wait()
    out_copy(bpc - 1).wait()


def kernel(x_lr, y_lr, x_hr, w1, w2, w3, s1, b1, s2, b2):
    b, c, hl, wl = x_lr.shape
    _, _, hh, wh = x_hr.shape
    assert c == 3 and (hl & (hl - 1)) == 0 and b % 2 == 0

    swn_t = jnp.asarray(_box_w_matrix_t(wl))
    uw_t = jnp.asarray(_resize_matrix(wh, wl).T)
    uh = jnp.asarray(_resize_matrix(hh, hl))
    s1c, b1c = s1.reshape(-1, 1), b1.reshape(-1, 1)
    s2c, b2c = s2.reshape(-1, 1), b2.reshape(-1, 1)

    n_cores = 2
    bpc = b // n_cores

    cspec = lambda a: pl.BlockSpec(a.shape, lambda i: (0,) * a.ndim)
    hbm = pl.BlockSpec(memory_space=pl.ANY)

    flops = b * (2 * 12 * hl * wl * (hl + wl)
                 + 2 * hl * wl * (6 * 32 + 32 * 32 + 32 * 3)
                 + 2 * 6 * hl * wl * wh
                 + 2 * 6 * hh * hl * wh + 2 * 3 * hh * wh)
    bytes_accessed = b * 4 * (2 * 3 * hl * wl + 2 * 3 * hh * wh) + 4 * hh * hl

    return pl.pallas_call(
        _gf_kernel,
        out_shape=jax.ShapeDtypeStruct((b, 3, hh, wh), jnp.float32),
        grid=(n_cores,),
        in_specs=[
            pl.BlockSpec((bpc, 3, hl, wl), lambda i: (i, 0, 0, 0)),
            pl.BlockSpec((bpc, 3, hl, wl), lambda i: (i, 0, 0, 0)),
            cspec(swn_t), cspec(uw_t), cspec(uh),
            cspec(w1), cspec(s1c), cspec(b1c),
            cspec(w2), cspec(s2c), cspec(b2c),
            cspec(w3),
            hbm,
        ],
        out_specs=hbm,
        scratch_shapes=[
            pltpu.VMEM((bpc, hl, 6 * wh), jnp.float32),
            pltpu.VMEM((min(bpc, 4), 3, hh, wh), jnp.float32),
            pltpu.VMEM((2, 3, hh, wh), jnp.float32),
            pltpu.SemaphoreType.DMA((min(bpc, 4),)),
            pltpu.SemaphoreType.DMA((2,)),
        ],
        compiler_params=pltpu.CompilerParams(
            dimension_semantics=("parallel",),
            vmem_limit_bytes=57 * 1024 * 1024,
        ),
        cost_estimate=pl.CostEstimate(flops=flops, transcendentals=0,
                                      bytes_accessed=bytes_accessed),
    )(x_lr, y_lr, swn_t, uw_t, uh, w1, s1c, b1c, w2, s2c, b2c, w3, x_hr)


# speedup vs baseline: 1.2561x; 1.0454x over previous
import numpy as np
import jax
import jax.numpy as jnp
from jax.experimental import pallas as pl
from jax.experimental.pallas import tpu as pltpu


def _box_w_matrix_t(n: int) -> np.ndarray:
    idx = np.arange(n)
    taps = (np.abs(idx[:, None] - idx[None, :]) <= 1).astype(np.float32)
    return np.ascontiguousarray((taps / taps.sum(axis=1, keepdims=True)).T)


def _resize_matrix(out_n: int, in_n: int) -> np.ndarray:
    if out_n == 1:
        m = np.zeros((1, in_n), np.float32)
        m[0, 0] = 1.0
        return m
    src = np.arange(out_n, dtype=np.float32) * (in_n - 1) / (out_n - 1)
    lo = np.clip(np.floor(src).astype(np.int32), 0, in_n - 1)
    hi = np.minimum(lo + 1, in_n - 1)
    t = src - lo.astype(np.float32)
    m = np.zeros((out_n, in_n), np.float32)
    np.add.at(m, (np.arange(out_n), lo), 1.0 - t)
    np.add.at(m, (np.arange(out_n), hi), t)
    return m


def _coeff_planes(x, y, swn_t, uw_t, w1, s1, b1, w2, s2, b2, w3, wh):
    hl, wl = x.shape[1], x.shape[2]
    p12 = jnp.concatenate([x, y, x * y, x * x], axis=0).reshape(12 * hl, wl)

    qw = jnp.dot(p12, swn_t, preferred_element_type=jnp.float32)
    r = jax.lax.broadcasted_iota(jnp.int32, qw.shape, 0) & (hl - 1)
    dn = pltpu.roll(qw, 1, axis=0)
    up = pltpu.roll(qw, 12 * hl - 1, axis=0)
    ssum = (qw + jnp.where(r == 0, 0.0, dn)
            + jnp.where(r == hl - 1, 0.0, up))
    nh = jnp.where(r == 0, 0.5, jnp.where(r == hl - 1, 0.5, 1.0 / 3.0))
    box3 = (ssum * nh).reshape(12, hl, wl)

    mx = box3[0:3]
    my = box3[3:6]
    cov = box3[6:9] - mx * my
    var = box3[9:12] - mx * mx

    feats = jnp.concatenate([cov, var], axis=0).reshape(6, hl * wl)
    h = jnp.dot(w1, feats, preferred_element_type=jnp.float32)
    h = jnp.maximum(h * s1 + b1, 0.0)
    h = jnp.dot(w2, h, preferred_element_type=jnp.float32)
    h = jnp.maximum(h * s2 + b2, 0.0)
    a = jnp.dot(w3, h, preferred_element_type=jnp.float32).reshape(3, hl, wl)
    bb = my - a * mx

    ab = jnp.concatenate([a, bb], axis=0).reshape(6 * hl, wl)
    return jnp.dot(ab, uw_t, preferred_element_type=jnp.float32)


def _gf_kernel(x_lr_ref, y_lr_ref, swn_t_ref, uw_t_ref, uh_ref,
               w1_ref, s1_ref, b1_ref, w2_ref, s2_ref, b2_ref, w3_ref,
               x_hr_hbm, out_hbm, planes_ref, xbuf, obuf, in_sem, out_sem):
    bpc = x_lr_ref.shape[0]
    hl = x_lr_ref.shape[2]
    hh, wh = out_hbm.shape[2], out_hbm.shape[3]
    core = pl.program_id(0)
    gb0 = core * bpc

    in_depth = min(bpc, 4)

    def in_copy(k):
        return pltpu.make_async_copy(
            x_hr_hbm.at[gb0 + k], xbuf.at[k % in_depth], in_sem.at[k % in_depth])

    hh2 = hh // 2

    def out_copy(k, h):
        slot = (2 * k + h) % 4
        return pltpu.make_async_copy(
            obuf.at[slot],
            out_hbm.at[gb0 + k, :, pl.ds(h * hh2, hh2), :],
            out_sem.at[slot])

    for k in range(in_depth):
        in_copy(k).start()
    for k in range(bpc):
        m6 = _coeff_planes(x_lr_ref[k], y_lr_ref[k],
                           swn_t_ref[...], uw_t_ref[...],
                           w1_ref[...], s1_ref[...], b1_ref[...],
                           w2_ref[...], s2_ref[...], b2_ref[...],
                           w3_ref[...], wh)
        for c in range(3):
            planes_ref[k, :, (2 * c) * wh:(2 * c + 1) * wh] = \
                m6[c * hl:(c + 1) * hl]
            planes_ref[k, :, (2 * c + 1) * wh:(2 * c + 2) * wh] = \
                m6[(3 + c) * hl:(4 + c) * hl]

    uh_blk = uh_ref[...]
    for k in range(bpc):
        s = k % in_depth
        in_copy(k).wait()
        for h in range(2):
            slot = (2 * k + h) % 4
            if 2 * k + h >= 4:
                out_copy(k - 2, h).wait()
            for c in range(3):
                pc = planes_ref[k, :, (2 * c) * wh:(2 * c + 2) * wh]
                m = jnp.dot(uh_blk[h * hh2:(h + 1) * hh2], pc,
                            preferred_element_type=jnp.float32)
                obuf[slot, c] = (m[:, :wh]
                                 * xbuf[s, c, h * hh2:(h + 1) * hh2] + m[:, wh:])
            out_copy(k, h).start()
        if k + in_depth < bpc:
            in_copy(k + in_depth).start()
    for k in range(max(bpc - 2, 0), bpc):
        out_copy(k, 0).wait()
        out_copy(k, 1).wait()


def kernel(x_lr, y_lr, x_hr, w1, w2, w3, s1, b1, s2, b2):
    b, c, hl, wl = x_lr.shape
    _, _, hh, wh = x_hr.shape
    assert c == 3 and (hl & (hl - 1)) == 0 and b % 2 == 0

    swn_t = jnp.asarray(_box_w_matrix_t(wl))
    uw_t = jnp.asarray(_resize_matrix(wh, wl).T)
    uh = jnp.asarray(_resize_matrix(hh, hl))
    s1c, b1c = s1.reshape(-1, 1), b1.reshape(-1, 1)
    s2c, b2c = s2.reshape(-1, 1), b2.reshape(-1, 1)

    n_cores = 2
    bpc = b // n_cores

    cspec = lambda a: pl.BlockSpec(a.shape, lambda i: (0,) * a.ndim)
    hbm = pl.BlockSpec(memory_space=pl.ANY)

    flops = b * (2 * 12 * hl * wl * (hl + wl)
                 + 2 * hl * wl * (6 * 32 + 32 * 32 + 32 * 3)
                 + 2 * 6 * hl * wl * wh
                 + 2 * 6 * hh * hl * wh + 2 * 3 * hh * wh)
    bytes_accessed = b * 4 * (2 * 3 * hl * wl + 2 * 3 * hh * wh) + 4 * hh * hl

    return pl.pallas_call(
        _gf_kernel,
        out_shape=jax.ShapeDtypeStruct((b, 3, hh, wh), jnp.float32),
        grid=(n_cores,),
        in_specs=[
            pl.BlockSpec((bpc, 3, hl, wl), lambda i: (i, 0, 0, 0)),
            pl.BlockSpec((bpc, 3, hl, wl), lambda i: (i, 0, 0, 0)),
            cspec(swn_t), cspec(uw_t), cspec(uh),
            cspec(w1), cspec(s1c), cspec(b1c),
            cspec(w2), cspec(s2c), cspec(b2c),
            cspec(w3),
            hbm,
        ],
        out_specs=hbm,
        scratch_shapes=[
            pltpu.VMEM((bpc, hl, 6 * wh), jnp.float32),
            pltpu.VMEM((min(bpc, 4), 3, hh, wh), jnp.float32),
            pltpu.VMEM((4, 3, hh // 2, wh), jnp.float32),
            pltpu.SemaphoreType.DMA((min(bpc, 4),)),
            pltpu.SemaphoreType.DMA((4,)),
        ],
        compiler_params=pltpu.CompilerParams(
            dimension_semantics=("parallel",),
            vmem_limit_bytes=57 * 1024 * 1024,
        ),
        cost_estimate=pl.CostEstimate(flops=flops, transcendentals=0,
                                      bytes_accessed=bytes_accessed),
    )(x_lr, y_lr, swn_t, uw_t, uh, w1, s1c, b1c, w2, s2c, b2c, w3, x_hr)
